# baseline (speedup 1.0000x reference)
import jax
import jax.numpy as jnp
from jax.experimental import pallas as pl

N_POINTS = 4096
N_CLUSTERS = 512
N_INPUT = 64
BLOCK_COLS = 2048

_DIMS = (((1,), (1,)), ((), ()))


def _kmeans_assign_kernel(c_ref, x_ref, out_ref):
    c = c_ref[...]
    x = x_ref[...]
    scores = jax.lax.dot_general(
        c, x, dimension_numbers=_DIMS, preferred_element_type=jnp.float32,
        precision=jax.lax.Precision.HIGHEST,
    )
    cn = jnp.sum(c * c, axis=1, keepdims=True)
    dist = cn - 2.0 * scores
    m = jnp.min(dist, axis=0, keepdims=True)
    idx = jax.lax.broadcasted_iota(jnp.int32, dist.shape, 0)
    cand = jnp.where(dist == m, idx, N_CLUSTERS)
    out_ref[...] = jnp.min(cand, axis=0, keepdims=True)


def kernel(x, centers):
    out = pl.pallas_call(
        _kmeans_assign_kernel,
        grid=(N_POINTS // BLOCK_COLS,),
        in_specs=[
            pl.BlockSpec((N_CLUSTERS, N_INPUT), lambda i: (0, 0)),
            pl.BlockSpec((BLOCK_COLS, N_INPUT), lambda i: (i, 0)),
        ],
        out_specs=pl.BlockSpec((1, BLOCK_COLS), lambda i: (0, i)),
        out_shape=jax.ShapeDtypeStruct((1, N_POINTS), jnp.int32),
    )(centers, x)
    return out.reshape(N_POINTS)
